# baseline (speedup 1.0000x reference)
import functools

import jax
import jax.numpy as jnp
import numpy as np
from jax.experimental import pallas as pl
from jax.experimental.pallas import tpu as pltpu


def _conv3x3_sep(xb, w3_ref, masks, W):
    GP = xb.shape[-1]
    t3 = jnp.concatenate([pltpu.roll(xb, 1, axis=1) * masks[3], xb,
                          pltpu.roll(xb, GP - 1, axis=1) * masks[5]], axis=0)
    s = jnp.dot(w3_ref[...], t3, preferred_element_type=jnp.float32)
    Cout = s.shape[0] // 3
    sm = s[:Cout].astype(jnp.bfloat16)
    sp = s[2 * Cout:].astype(jnp.bfloat16)
    return (s[Cout:2 * Cout]
            + pltpu.roll(sm, W, axis=1) * masks[1]
            + pltpu.roll(sp, GP - W, axis=1) * masks[7])


def _dw3x3(xb, wc_ref, base, masks, W):
    P = xb.shape[-1]
    tm = pltpu.roll(xb, 1, axis=1) * masks[3]
    tp = pltpu.roll(xb, P - 1, axis=1) * masks[5]
    s = [wc_ref[base + 3 * r] * tm
         + wc_ref[base + 3 * r + 1] * xb
         + wc_ref[base + 3 * r + 2] * tp for r in range(3)]
    return (s[1]
            + pltpu.roll(s[0], W, axis=1) * masks[1]
            + pltpu.roll(s[2], P - W, axis=1) * masks[7])


def _ghost_kernel(x_ref, mask_ref,
                  wst_ref, bst_ref, wgc_ref, bgc_ref,
                  wa_ref, w1c_ref, w2_ref, w2c_ref, wy_ref, bs_ref,
                  wout_ref, bout_ref, o_ref, *, W, L, G, P):
    masks = [None if k == 4 else mask_ref[k] for k in range(9)]

    xb = jnp.concatenate([x_ref[g].astype(jnp.bfloat16) for g in range(G)],
                         axis=1)

    h = _conv3x3_sep(xb, wst_ref, masks, W) + bst_ref[...]
    x = (_conv3x3_sep(h.astype(jnp.bfloat16), wgc_ref, masks, W)
         + bgc_ref[...])

    for l in range(L):
        xb_l = x.astype(jnp.bfloat16)
        t = jnp.dot(wa_ref[l], xb_l, preferred_element_type=jnp.float32)
        p1b = jnp.maximum(t, 0.0).astype(jnp.bfloat16)

        c1 = jnp.maximum(_dw3x3(p1b, w1c_ref, l * 9, masks, W), 0.0)
        s1 = jnp.concatenate([p1b, c1], axis=0)
        p2b = jnp.dot(w2_ref[l], s1,
                      preferred_element_type=jnp.float32).astype(jnp.bfloat16)

        c2 = _dw3x3(p2b, w2c_ref, l * 9, masks, W)
        s2 = jnp.concatenate([p2b, c2], axis=0)
        y = jnp.dot(wy_ref[l], (s2 + x).astype(jnp.bfloat16),
                    preferred_element_type=jnp.float32)

        x = y + bs_ref[l] + x

    out = jnp.dot(wout_ref[...], x.astype(jnp.bfloat16),
                  preferred_element_type=jnp.float32)
    res = out + bout_ref[...] + h
    for g in range(G):
        o_ref[g] = res[:, g * P:(g + 1) * P]


def kernel(x, start_w, start_b, ghostconv_w, ghostconv_b, out_w, out_b,
           g1pw_0, g1cw_0, g2pw_0, g2cw_0, specw_0, specb_0,
           g1pw_1, g1cw_1, g2pw_1, g2cw_1, specw_1, specb_1,
           g1pw_2, g1cw_2, g2pw_2, g2cw_2, specw_2, specb_2,
           g1pw_3, g1cw_3, g2pw_3, g2cw_3, specw_3, specb_3,
           g1pw_4, g1cw_4, g2pw_4, g2cw_4, specw_4, specb_4):
    g1pw = [g1pw_0, g1pw_1, g1pw_2, g1pw_3, g1pw_4]
    g1cw = [g1cw_0, g1cw_1, g1cw_2, g1cw_3, g1cw_4]
    g2pw = [g2pw_0, g2pw_1, g2pw_2, g2pw_3, g2pw_4]
    g2cw = [g2cw_0, g2cw_1, g2cw_2, g2cw_3, g2cw_4]
    specw = [specw_0, specw_1, specw_2, specw_3, specw_4]
    specb = [specb_0, specb_1, specb_2, specb_3, specb_4]
    L = len(g1pw)

    N, Cin, H, W = x.shape
    P = H * W
    G = 2 if N % 2 == 0 else 1
    Co = start_w.shape[3]
    Cg = ghostconv_w.shape[3]
    I1 = g1pw_0.shape[1]
    I2 = g2pw_0.shape[1]
    bf = jnp.bfloat16

    pix = np.arange(P)
    row, col = pix // W, pix % W
    m = np.ones((9, 1, P), np.float32)
    k = 0
    for ry in (-1, 0, 1):
        for rx in (-1, 0, 1):
            valid = ((row + ry >= 0) & (row + ry < H) &
                     (col + rx >= 0) & (col + rx < W))
            m[k, 0, :] = valid.astype(np.float32)
            k += 1
    masks = jnp.asarray(np.tile(m, (1, 1, G)), bf)

    def colv(b):
        return jnp.asarray(b, jnp.float32).reshape(-1, 1)

    def c3blk(w):
        return jnp.concatenate(
            [jnp.concatenate([w[r, c].T for c in range(3)], axis=1)
             for r in range(3)], axis=0).astype(bf)

    consts = [
        masks,
        c3blk(start_w), colv(start_b),
        c3blk(ghostconv_w), colv(ghostconv_b),
        jnp.stack([g1pw[l].T for l in range(L)]).astype(bf),
        jnp.concatenate([g1cw[l].reshape(9, I1, 1)
                         for l in range(L)], axis=0).astype(bf),
        jnp.stack([g2pw[l].T for l in range(L)]).astype(bf),
        jnp.concatenate([g2cw[l].reshape(9, I2, 1)
                         for l in range(L)], axis=0).astype(bf),
        jnp.stack([specw[l].T for l in range(L)]).astype(bf),
        jnp.stack([colv(specb[l]) for l in range(L)]),
        out_w.T.astype(bf),
        colv(out_b),
    ]

    xf = x.reshape(N, Cin, P).astype(jnp.float32)

    def resident(a):
        zeros = (0,) * a.ndim
        return pl.BlockSpec(a.shape, lambda n, z=zeros: z)

    fn = functools.partial(_ghost_kernel, W=W, L=L, G=G, P=P)
    out = pl.pallas_call(
        fn,
        out_shape=jax.ShapeDtypeStruct((N, Co, P), jnp.float32),
        grid=(N // G,),
        in_specs=[pl.BlockSpec((G, Cin, P), lambda n: (n, 0, 0))]
                + [resident(a) for a in consts],
        out_specs=pl.BlockSpec((G, Co, P), lambda n: (n, 0, 0)),
        compiler_params=pltpu.CompilerParams(
            dimension_semantics=("parallel",),
            vmem_limit_bytes=64 * 1024 * 1024,
        ),
    )(xf, *consts)
    return out.reshape(N, Co, H, W)


# speedup vs baseline: 1.0521x; 1.0521x over previous
import functools

import jax
import jax.numpy as jnp
import numpy as np
from jax.experimental import pallas as pl
from jax.experimental.pallas import tpu as pltpu


def _conv3x3_sep(xb, w3_ref, masks, W):
    GP = xb.shape[-1]
    t3 = jnp.concatenate([pltpu.roll(xb, 1, axis=1) * masks[3], xb,
                          pltpu.roll(xb, GP - 1, axis=1) * masks[5]], axis=0)
    s = jnp.dot(w3_ref[...], t3, preferred_element_type=jnp.float32)
    Cout = s.shape[0] // 3
    sm = s[:Cout].astype(jnp.bfloat16)
    sp = s[2 * Cout:].astype(jnp.bfloat16)
    return (s[Cout:2 * Cout]
            + pltpu.roll(sm, W, axis=1) * masks[1]
            + pltpu.roll(sp, GP - W, axis=1) * masks[7])


def _dw3x3(xb, wc_ref, base, masks, W):
    P = xb.shape[-1]
    tm = pltpu.roll(xb, 1, axis=1) * masks[3]
    tp = pltpu.roll(xb, P - 1, axis=1) * masks[5]
    s = [wc_ref[base + 3 * r] * tm
         + wc_ref[base + 3 * r + 1] * xb
         + wc_ref[base + 3 * r + 2] * tp for r in range(3)]
    return (s[1]
            + pltpu.roll(s[0], W, axis=1) * masks[1]
            + pltpu.roll(s[2], P - W, axis=1) * masks[7])


def _ghost_kernel(x_ref, mask_ref,
                  wst_ref, bst_ref, wgc_ref, bgc_ref,
                  wa_ref, w1c_ref, w2_ref, w2c_ref, wy_ref, bs_ref,
                  wout_ref, bout_ref, o_ref, *, W, L, G, P):
    masks = [None if k == 4 else mask_ref[k] for k in range(9)]

    xb = jnp.concatenate([x_ref[g].astype(jnp.bfloat16) for g in range(G)],
                         axis=1)

    h = _conv3x3_sep(xb, wst_ref, masks, W) + bst_ref[...]
    x = (_conv3x3_sep(h.astype(jnp.bfloat16), wgc_ref, masks, W)
         + bgc_ref[...])

    for l in range(L):
        xb_l = x.astype(jnp.bfloat16)
        t = jnp.dot(wa_ref[l], xb_l, preferred_element_type=jnp.float32)
        p1b = jnp.maximum(t, 0.0).astype(jnp.bfloat16)

        c1 = jnp.maximum(_dw3x3(p1b, w1c_ref, l * 9, masks, W), 0.0)
        s1 = jnp.concatenate([p1b, c1], axis=0)
        p2b = jnp.dot(w2_ref[l], s1,
                      preferred_element_type=jnp.float32).astype(jnp.bfloat16)

        c2 = _dw3x3(p2b, w2c_ref, l * 9, masks, W)
        s2 = jnp.concatenate([p2b, c2], axis=0)
        y = jnp.dot(wy_ref[l], (s2 + x).astype(jnp.bfloat16),
                    preferred_element_type=jnp.float32)

        x = y + bs_ref[l] + x

    out = jnp.dot(wout_ref[...], x.astype(jnp.bfloat16),
                  preferred_element_type=jnp.float32)
    res = out + bout_ref[...] + h
    for g in range(G):
        o_ref[g] = res[:, g * P:(g + 1) * P]


def kernel(x, start_w, start_b, ghostconv_w, ghostconv_b, out_w, out_b,
           g1pw_0, g1cw_0, g2pw_0, g2cw_0, specw_0, specb_0,
           g1pw_1, g1cw_1, g2pw_1, g2cw_1, specw_1, specb_1,
           g1pw_2, g1cw_2, g2pw_2, g2cw_2, specw_2, specb_2,
           g1pw_3, g1cw_3, g2pw_3, g2cw_3, specw_3, specb_3,
           g1pw_4, g1cw_4, g2pw_4, g2cw_4, specw_4, specb_4):
    g1pw = [g1pw_0, g1pw_1, g1pw_2, g1pw_3, g1pw_4]
    g1cw = [g1cw_0, g1cw_1, g1cw_2, g1cw_3, g1cw_4]
    g2pw = [g2pw_0, g2pw_1, g2pw_2, g2pw_3, g2pw_4]
    g2cw = [g2cw_0, g2cw_1, g2cw_2, g2cw_3, g2cw_4]
    specw = [specw_0, specw_1, specw_2, specw_3, specw_4]
    specb = [specb_0, specb_1, specb_2, specb_3, specb_4]
    L = len(g1pw)

    N, Cin, H, W = x.shape
    P = H * W
    G = 4 if N % 4 == 0 else (2 if N % 2 == 0 else 1)
    Co = start_w.shape[3]
    Cg = ghostconv_w.shape[3]
    I1 = g1pw_0.shape[1]
    I2 = g2pw_0.shape[1]
    bf = jnp.bfloat16

    pix = np.arange(P)
    row, col = pix // W, pix % W
    m = np.ones((9, 1, P), np.float32)
    k = 0
    for ry in (-1, 0, 1):
        for rx in (-1, 0, 1):
            valid = ((row + ry >= 0) & (row + ry < H) &
                     (col + rx >= 0) & (col + rx < W))
            m[k, 0, :] = valid.astype(np.float32)
            k += 1
    masks = jnp.asarray(np.tile(m, (1, 1, G)), bf)

    def colv(b):
        return jnp.asarray(b, jnp.float32).reshape(-1, 1)

    def c3blk(w):
        return jnp.concatenate(
            [jnp.concatenate([w[r, c].T for c in range(3)], axis=1)
             for r in range(3)], axis=0).astype(bf)

    consts = [
        masks,
        c3blk(start_w), colv(start_b),
        c3blk(ghostconv_w), colv(ghostconv_b),
        jnp.stack([g1pw[l].T for l in range(L)]).astype(bf),
        jnp.concatenate([g1cw[l].reshape(9, I1, 1)
                         for l in range(L)], axis=0).astype(bf),
        jnp.stack([g2pw[l].T for l in range(L)]).astype(bf),
        jnp.concatenate([g2cw[l].reshape(9, I2, 1)
                         for l in range(L)], axis=0).astype(bf),
        jnp.stack([specw[l].T for l in range(L)]).astype(bf),
        jnp.stack([colv(specb[l]) for l in range(L)]),
        out_w.T.astype(bf),
        colv(out_b),
    ]

    xf = x.reshape(N, Cin, P).astype(jnp.float32)

    def resident(a):
        zeros = (0,) * a.ndim
        return pl.BlockSpec(a.shape, lambda n, z=zeros: z)

    fn = functools.partial(_ghost_kernel, W=W, L=L, G=G, P=P)
    out = pl.pallas_call(
        fn,
        out_shape=jax.ShapeDtypeStruct((N, Co, P), jnp.float32),
        grid=(N // G,),
        in_specs=[pl.BlockSpec((G, Cin, P), lambda n: (n, 0, 0))]
                + [resident(a) for a in consts],
        out_specs=pl.BlockSpec((G, Co, P), lambda n: (n, 0, 0)),
        compiler_params=pltpu.CompilerParams(
            dimension_semantics=("parallel",),
            vmem_limit_bytes=64 * 1024 * 1024,
        ),
    )(xf, *consts)
    return out.reshape(N, Co, H, W)
